# speedup vs baseline: 1.5305x; 1.5305x over previous
import functools

import jax
import jax.numpy as jnp
from jax import lax
from jax.experimental import pallas as pl
from jax.experimental.pallas import tpu as pltpu
from jax.experimental.pallas import tpu_sc as plsc

M = 10000
K = 16
D = 256
L = 16
NC = 2
NS = 16
NW = NC * NS
MP = 10240
RPW = MP // NW
C = 8
CK = C * K
NCH = RPW // C
DBLK = D // L


def _compute_chunk(rows_buf, out_buf):
    def blk(t, carry):
        c = t >> 4
        col = (t & 15) * L
        r0 = c * K
        acc = rows_buf[r0, pl.ds(col, L)]
        for k in range(1, K):
            acc = jnp.maximum(acc, rows_buf[r0 + k, pl.ds(col, L)])
        out_buf[c, pl.ds(col, L)] = acc
        return carry

    lax.fori_loop(0, C * DBLK, blk, 0)


@functools.partial(
    pl.kernel,
    mesh=plsc.VectorSubcoreMesh(core_axis_name="c", subcore_axis_name="s"),
    out_type=jax.ShapeDtypeStruct((MP, D), jnp.float32),
    scratch_types=[
        pltpu.VMEM((RPW * K,), jnp.int32),
        pltpu.VMEM((CK, D), jnp.float32),
        pltpu.VMEM((CK, D), jnp.float32),
        pltpu.VMEM((C, D), jnp.float32),
        pltpu.SemaphoreType.DMA,
        pltpu.SemaphoreType.DMA,
    ],
)
def _max_pool_sc(x_hbm, nbr_hbm, out_hbm, idx_v, rows_a, rows_b, out_v,
                 sem_a, sem_b):
    cid = lax.axis_index("c")
    sid = lax.axis_index("s")
    wid = sid * NC + cid
    base = wid * RPW

    pltpu.sync_copy(nbr_hbm.at[pl.ds(base * K, RPW * K)], idx_v)

    def gather(chunk, buf, sem):
        idx = idx_v.at[pl.ds(chunk * CK, CK)]
        return pltpu.make_async_copy(x_hbm.at[idx], buf, sem)

    gather(0, rows_a, sem_a).start()
    gather(1, rows_b, sem_b).start()

    def step(t, carry):
        def half(chunk, buf, sem, next_chunk):
            gather(chunk, buf, sem).wait()
            _compute_chunk(buf, out_v)
            pltpu.sync_copy(out_v, out_hbm.at[pl.ds(base + chunk * C, C)])

            @pl.when(next_chunk < NCH)
            def _():
                gather(next_chunk, buf, sem).start()

        half(2 * t, rows_a, sem_a, 2 * t + 2)
        half(2 * t + 1, rows_b, sem_b, 2 * t + 3)
        return carry

    lax.fori_loop(0, NCH // 2, step, 0)


def kernel(x_feats, neighbor_indices):
    nbr = neighbor_indices.astype(jnp.int32)
    pad = jnp.zeros((MP - M, K), jnp.int32)
    nbr_flat = jnp.concatenate([nbr, pad], axis=0).reshape(MP * K)
    out = _max_pool_sc(x_feats, nbr_flat)
    return out[:M]


# speedup vs baseline: 1.5748x; 1.0289x over previous
import functools

import jax
import jax.numpy as jnp
from jax import lax
from jax.experimental import pallas as pl
from jax.experimental.pallas import tpu as pltpu
from jax.experimental.pallas import tpu_sc as plsc

M = 10000
K = 16
D = 256
L = 16
NC = 2
NS = 16
NW = NC * NS
MP = 10240
RPW = MP // NW
C = 4
CK = C * K
NCH = RPW // C
NBUF = 4
DBLK = D // L


def _compute_chunk(rows_buf, out_buf):
    @plsc.parallel_loop(0, C * DBLK, unroll=4)
    def blk(t):
        c = t >> 4
        col = (t & 15) * L
        r0 = c * K
        acc = rows_buf[r0, pl.ds(col, L)]
        for k in range(1, K):
            acc = jnp.maximum(acc, rows_buf[r0 + k, pl.ds(col, L)])
        out_buf[c, pl.ds(col, L)] = acc


@functools.partial(
    pl.kernel,
    mesh=plsc.VectorSubcoreMesh(core_axis_name="c", subcore_axis_name="s"),
    out_type=jax.ShapeDtypeStruct((MP, D), jnp.float32),
    scratch_types=[
        pltpu.VMEM((RPW * K,), jnp.int32),
        [pltpu.VMEM((CK, D), jnp.float32)] * NBUF,
        pltpu.VMEM((C, D), jnp.float32),
        [pltpu.SemaphoreType.DMA] * NBUF,
    ],
)
def _max_pool_sc(x_hbm, nbr_hbm, out_hbm, idx_v, rows, out_v, sems):
    cid = lax.axis_index("c")
    sid = lax.axis_index("s")
    wid = sid * NC + cid
    base = wid * RPW

    pltpu.sync_copy(nbr_hbm.at[pl.ds(base * K, RPW * K)], idx_v)

    def gather(chunk, b):
        idx = idx_v.at[pl.ds(chunk * CK, CK)]
        return pltpu.make_async_copy(x_hbm.at[idx], rows[b], sems[b])

    for b in range(NBUF):
        gather(b, b).start()

    def step(t, carry):
        for b in range(NBUF):
            chunk = NBUF * t + b
            gather(chunk, b).wait()
            _compute_chunk(rows[b], out_v)
            pltpu.sync_copy(out_v, out_hbm.at[pl.ds(base + chunk * C, C)])

            @pl.when(chunk + NBUF < NCH)
            def _():
                gather(chunk + NBUF, b).start()

        return carry

    lax.fori_loop(0, NCH // NBUF, step, 0)


def kernel(x_feats, neighbor_indices):
    nbr = neighbor_indices.astype(jnp.int32)
    pad = jnp.zeros((MP - M, K), jnp.int32)
    nbr_flat = jnp.concatenate([nbr, pad], axis=0).reshape(MP * K)
    out = _max_pool_sc(x_feats, nbr_flat)
    return out[:M]
